# baseline (speedup 1.0000x reference)
import functools

import jax
import jax.numpy as jnp
from jax import lax
from jax.experimental import pallas as pl
from jax.experimental.pallas import tpu as pltpu
from jax.experimental.pallas import tpu_sc as plsc

B = 1024
C = 8
EMBED = 128
N = B * C
NC, NS = 2, 16
NW = NC * NS
RPW = N // NW
CHPW = RPW // C
NSTREAM = 4
RPS = RPW // NSTREAM
NCHUNK = RPW // 16
CPS = NCHUNK // NSTREAM
BUFW = 2 * 16 * 17


def _log_sigmoid(s):
    t = jnp.exp(-jnp.abs(s))
    u = t / (t + 2.0)
    u2 = u * u
    p = u * (2.0 + u2 * (2.0 / 3.0 + u2 * (2.0 / 5.0 + u2 * (2.0 / 7.0))))
    return jnp.minimum(s, 0.0) - p


@functools.lru_cache(maxsize=None)
def _make_sc_kernel():
    mesh = plsc.VectorSubcoreMesh(core_axis_name="c", subcore_axis_name="s")

    @functools.partial(
        pl.kernel,
        mesh=mesh,
        out_type=jax.ShapeDtypeStruct((N,), jnp.float32),
        compiler_params=pltpu.CompilerParams(needs_layout_passes=False),
        scratch_types=[
            pltpu.VMEM((CHPW, C), jnp.int32),
            pltpu.VMEM((RPW,), jnp.int32),
            pltpu.VMEM((RPW, EMBED), jnp.float32),
            pltpu.VMEM((BUFW,), jnp.float32),
            pltpu.VMEM((RPW,), jnp.float32),
            pltpu.VMEM((RPW,), jnp.float32),
            pltpu.SemaphoreType.DMA,
            pltpu.SemaphoreType.DMA,
            pltpu.SemaphoreType.DMA,
            pltpu.SemaphoreType.DMA,
        ],
    )
    def sc_k(chords_hbm, table_hbm, out_hbm, idx2_v, idx_v, rows_v, buf_v,
             cnt_v, out_v, sem0, sem1, sem2, sem3):
        wid = lax.axis_index("s") * NC + lax.axis_index("c")
        base = wid * RPW
        pltpu.sync_copy(chords_hbm.at[pl.ds(wid * CHPW, CHPW)], idx2_v)

        iota = lax.iota(jnp.int32, 16)
        zeros = jnp.zeros((16,), jnp.float32)

        hi = lax.shift_right_logical(iota, 3)
        lo = iota & 7
        for t in range(16):
            idx_v[pl.ds(t * 16, 16)] = plsc.load_gather(
                idx2_v, [hi + 2 * t, lo])

        sems = [sem0, sem1, sem2, sem3]
        copies = [
            pltpu.async_copy(table_hbm.at[idx_v.at[pl.ds(j * RPS, RPS)]],
                             rows_v.at[pl.ds(j * RPS, RPS)], sems[j])
            for j in range(NSTREAM)
        ]

        def cnt_body(c, _):
            v = idx_v[pl.ds(c * 16, 16)]
            cnt = jnp.ones((16,), jnp.int32)
            cbase = c * 16 + (iota & ~7)
            for k in range(1, C):
                w = plsc.load_gather(idx_v, [cbase + ((iota + k) & 7)])
                cnt = cnt + jnp.where(w == v, 1, 0)
            cnt_v[pl.ds(c * 16, 16)] = cnt.astype(jnp.float32)
            return 0

        lax.fori_loop(0, NCHUNK, cnt_body, 0)

        def chunk_body(c, _):
            for bb in range(2):
                rb = c * 16 + bb * 8
                accd = [zeros] * C
                accq = [zeros] * C
                for k in range(EMBED // 16):
                    e = [rows_v[rb + i, pl.ds(k * 16, 16)]
                         for i in range(C)]
                    sk = ((e[0] + e[1]) + (e[2] + e[3])) + (
                        (e[4] + e[5]) + (e[6] + e[7]))
                    for i in range(C):
                        accd[i] = accd[i] + e[i] * sk
                        accq[i] = accq[i] + e[i] * e[i]
                for i in range(C):
                    buf_v[pl.ds((bb * 8 + i) * 17, 16)] = accd[i]
                    buf_v[pl.ds((16 + bb * 8 + i) * 17, 16)] = accq[i]
            dpk = zeros
            qpk = zeros
            for k in range(16):
                dpk = dpk + plsc.load_gather(buf_v, [iota * 17 + k])
                qpk = qpk + plsc.load_gather(buf_v, [iota * 17 + (272 + k)])

            nf = cnt_v[pl.ds(c * 16, 16)]
            out_v[pl.ds(c * 16, 16)] = (dpk - nf * qpk) * (1.0 / 7.0)
            return 0

        for j in range(NSTREAM):
            copies[j].wait()
        lax.fori_loop(0, NCHUNK, chunk_body, 0)

        def sig_body(c, _):
            out_v[pl.ds(c * 16, 16)] = _log_sigmoid(out_v[pl.ds(c * 16, 16)])
            return 0

        lax.fori_loop(0, NCHUNK, sig_body, 0)

        pltpu.sync_copy(out_v, out_hbm.at[pl.ds(base, RPW)])

    return sc_k


def kernel(chords, weight):
    out = _make_sc_kernel()(chords, weight)
    return out.reshape(N, 1, 1)


# speedup vs baseline: 1.0103x; 1.0103x over previous
import functools

import jax
import jax.numpy as jnp
from jax import lax
from jax.experimental import pallas as pl
from jax.experimental.pallas import tpu as pltpu
from jax.experimental.pallas import tpu_sc as plsc

B = 1024
C = 8
EMBED = 128
N = B * C
NC, NS = 2, 16
NW = NC * NS
RPW = N // NW
CHPW = RPW // C
NSTREAM = 2
RPS = RPW // NSTREAM
NCHUNK = RPW // 16
CPS = NCHUNK // NSTREAM
BUFW = 2 * 16 * 17


def _log_sigmoid(s):
    t = jnp.exp(-jnp.abs(s))
    u = t / (t + 2.0)
    u2 = u * u
    p = u * (2.0 + u2 * (2.0 / 3.0 + u2 * (2.0 / 5.0 + u2 * (2.0 / 7.0))))
    return jnp.minimum(s, 0.0) - p


@functools.lru_cache(maxsize=None)
def _make_sc_kernel():
    mesh = plsc.VectorSubcoreMesh(core_axis_name="c", subcore_axis_name="s")

    @functools.partial(
        pl.kernel,
        mesh=mesh,
        out_type=jax.ShapeDtypeStruct((N,), jnp.float32),
        compiler_params=pltpu.CompilerParams(needs_layout_passes=False),
        scratch_types=[
            pltpu.VMEM((CHPW, C), jnp.int32),
            pltpu.VMEM((RPW,), jnp.int32),
            pltpu.VMEM((RPW, EMBED), jnp.float32),
            pltpu.VMEM((BUFW,), jnp.float32),
            pltpu.VMEM((RPW,), jnp.float32),
            pltpu.VMEM((RPW,), jnp.float32),
            pltpu.SemaphoreType.DMA,
            pltpu.SemaphoreType.DMA,
        ],
    )
    def sc_k(chords_hbm, table_hbm, out_hbm, idx2_v, idx_v, rows_v, buf_v,
             cnt_v, out_v, sem0, sem1):
        wid = lax.axis_index("s") * NC + lax.axis_index("c")
        base = wid * RPW
        pltpu.sync_copy(chords_hbm.at[pl.ds(wid * CHPW, CHPW)], idx2_v)

        iota = lax.iota(jnp.int32, 16)
        zeros = jnp.zeros((16,), jnp.float32)

        hi = lax.shift_right_logical(iota, 3)
        lo = iota & 7
        for t in range(16):
            idx_v[pl.ds(t * 16, 16)] = plsc.load_gather(
                idx2_v, [hi + 2 * t, lo])

        sems = [sem0, sem1]
        copies = [
            pltpu.async_copy(table_hbm.at[idx_v.at[pl.ds(j * RPS, RPS)]],
                             rows_v.at[pl.ds(j * RPS, RPS)], sems[j])
            for j in range(NSTREAM)
        ]

        def cnt_body(c, _):
            v = idx_v[pl.ds(c * 16, 16)]
            cnt = jnp.ones((16,), jnp.int32)
            cbase = c * 16 + (iota & ~7)
            for k in range(1, C):
                w = plsc.load_gather(idx_v, [cbase + ((iota + k) & 7)])
                cnt = cnt + jnp.where(w == v, 1, 0)
            cnt_v[pl.ds(c * 16, 16)] = cnt.astype(jnp.float32)
            return 0

        lax.fori_loop(0, NCHUNK, cnt_body, 0)

        def chunk_body(c, _):
            for bb in range(2):
                rb = c * 16 + bb * 8
                accd = [zeros] * C
                accq = [zeros] * C
                for k in range(EMBED // 16):
                    e = [rows_v[rb + i, pl.ds(k * 16, 16)]
                         for i in range(C)]
                    sk = ((e[0] + e[1]) + (e[2] + e[3])) + (
                        (e[4] + e[5]) + (e[6] + e[7]))
                    for i in range(C):
                        accd[i] = accd[i] + e[i] * sk
                        accq[i] = accq[i] + e[i] * e[i]
                for i in range(C):
                    buf_v[pl.ds((bb * 8 + i) * 17, 16)] = accd[i]
                    buf_v[pl.ds((16 + bb * 8 + i) * 17, 16)] = accq[i]
            dpk = zeros
            qpk = zeros
            for k in range(16):
                dpk = dpk + plsc.load_gather(buf_v, [iota * 17 + k])
                qpk = qpk + plsc.load_gather(buf_v, [iota * 17 + (272 + k)])

            nf = cnt_v[pl.ds(c * 16, 16)]
            s = (dpk - nf * qpk) * (1.0 / 7.0)
            out_v[pl.ds(c * 16, 16)] = _log_sigmoid(s)
            return 0

        for j in range(NSTREAM):
            copies[j].wait()
        lax.fori_loop(0, NCHUNK, chunk_body, 0)

        pltpu.sync_copy(out_v, out_hbm.at[pl.ds(base, RPW)])

    return sc_k


def kernel(chords, weight):
    out = _make_sc_kernel()(chords, weight)
    return out.reshape(N, 1, 1)
